# baseline (device time: 23058 ns/iter reference)
import jax
import jax.numpy as jnp
from jax import lax
from jax.experimental import pallas as pl
from jax.experimental.pallas import tpu as pltpu

N_DEV = 16
N_GRP = 4
G = N_DEV // N_GRP


def kernel(x, w_mat):
    m_per, k = x.shape
    _, n = w_mat.shape
    n_per = n // N_DEV

    def body(x_ref, w_ref, out_ref, wbuf, comm_ref, load_sems,
             send_sems, recv_sems):
        my = lax.axis_index("i")

        barrier = pltpu.get_barrier_semaphore()
        for p in range(N_DEV):
            @pl.when(my != p)
            def _():
                pl.semaphore_signal(
                    barrier, inc=1,
                    device_id=(p,), device_id_type=pl.DeviceIdType.MESH,
                )
        pl.semaphore_wait(barrier, N_DEV - 1)

        loads = []
        for jo in range(1, N_DEV + 1):
            tgt = lax.rem(my + jo, N_DEV)
            c, j = (jo - 1) // G, (jo - 1) % G
            cp = pltpu.make_async_copy(
                w_ref.at[:, pl.ds(tgt * n_per, n_per)],
                wbuf.at[c, :, pl.ds(j * n_per, n_per)],
                load_sems.at[jo - 1],
            )
            cp.start()
            loads.append(cp)

        x_mat = x_ref[:, :]
        sends = []
        for c in range(N_GRP):
            for j in range(G):
                loads[c * G + j].wait()
            grp = jnp.dot(x_mat, wbuf[c], preferred_element_type=jnp.float32)
            grp = grp * jax.nn.sigmoid(grp)
            comm_ref[c, :, :] = grp.astype(comm_ref.dtype)
            for j in range(G):
                jo = c * G + j + 1
                if jo == N_DEV:
                    out_ref[pl.ds(my * m_per, m_per), :] = (
                        comm_ref[c, :, pl.ds(j * n_per, n_per)])
                else:
                    tgt = lax.rem(my + jo, N_DEV)
                    rdma = pltpu.make_async_remote_copy(
                        src_ref=comm_ref.at[c, :, pl.ds(j * n_per, n_per)],
                        dst_ref=out_ref.at[pl.ds(my * m_per, m_per), :],
                        send_sem=send_sems.at[jo],
                        recv_sem=recv_sems.at[jo],
                        device_id=(tgt,),
                        device_id_type=pl.DeviceIdType.MESH,
                    )
                    rdma.start()
                    sends.append(rdma)

        for jo in range(1, N_DEV):
            src_dev = lax.rem(my + (N_DEV - jo), N_DEV)
            recv = pltpu.make_async_remote_copy(
                src_ref=comm_ref.at[0, :, pl.ds(0, n_per)],
                dst_ref=out_ref.at[pl.ds(src_dev * m_per, m_per), :],
                send_sem=send_sems.at[0],
                recv_sem=recv_sems.at[jo],
                device_id=(my,),
                device_id_type=pl.DeviceIdType.MESH,
            )
            recv.wait_recv()

        for rdma in sends:
            rdma.wait_send()

    out_shape = jax.ShapeDtypeStruct((N_DEV * m_per, n_per), jnp.bfloat16)
    return pl.pallas_call(
        body,
        out_shape=out_shape,
        in_specs=[
            pl.BlockSpec(memory_space=pltpu.VMEM),
            pl.BlockSpec(memory_space=pltpu.MemorySpace.HBM),
        ],
        out_specs=pl.BlockSpec(memory_space=pltpu.VMEM),
        scratch_shapes=[
            pltpu.VMEM((N_GRP, k, G * n_per), jnp.float32),
            pltpu.VMEM((N_GRP, m_per, G * n_per), jnp.bfloat16),
            pltpu.SemaphoreType.DMA((N_DEV,)),
            pltpu.SemaphoreType.DMA((N_DEV,)),
            pltpu.SemaphoreType.DMA((N_DEV,)),
        ],
        compiler_params=pltpu.CompilerParams(collective_id=0),
    )(x, w_mat)


# device time: 20769 ns/iter; 1.1102x vs baseline; 1.1102x over previous
import jax
import jax.numpy as jnp
from jax import lax
from jax.experimental import pallas as pl
from jax.experimental.pallas import tpu as pltpu

N_DEV = 16


def kernel(x, w_mat):
    m_per, k = x.shape
    _, n = w_mat.shape
    n_per = n // N_DEV

    def body(x_ref, w_ref, out_ref, y_ref, send_sems, recv_sems):
        my = lax.axis_index("i")

        barrier = pltpu.get_barrier_semaphore()
        for p in range(N_DEV):
            @pl.when(my != p)
            def _():
                pl.semaphore_signal(
                    barrier, inc=1,
                    device_id=(p,), device_id_type=pl.DeviceIdType.MESH,
                )
        pl.semaphore_wait(barrier, N_DEV - 1)

        y = jnp.dot(x_ref[:, :], w_ref[:, :], preferred_element_type=jnp.float32)
        y = y * jax.nn.sigmoid(y)
        y_ref[:, :] = y.astype(y_ref.dtype)

        out_ref[pl.ds(my * m_per, m_per), :] = y_ref[:, pl.ds(my * n_per, n_per)]

        sends = []
        for jo in range(1, N_DEV):
            tgt = lax.rem(my + jo, N_DEV)
            rdma = pltpu.make_async_remote_copy(
                src_ref=y_ref.at[:, pl.ds(tgt * n_per, n_per)],
                dst_ref=out_ref.at[pl.ds(my * m_per, m_per), :],
                send_sem=send_sems.at[jo],
                recv_sem=recv_sems.at[jo],
                device_id=(tgt,),
                device_id_type=pl.DeviceIdType.MESH,
            )
            rdma.start()
            sends.append(rdma)

        for jo in range(1, N_DEV):
            src_dev = lax.rem(my + (N_DEV - jo), N_DEV)
            recv = pltpu.make_async_remote_copy(
                src_ref=y_ref.at[:, pl.ds(0, n_per)],
                dst_ref=out_ref.at[pl.ds(src_dev * m_per, m_per), :],
                send_sem=send_sems.at[0],
                recv_sem=recv_sems.at[jo],
                device_id=(my,),
                device_id_type=pl.DeviceIdType.MESH,
            )
            recv.wait_recv()

        for rdma in sends:
            rdma.wait_send()

    out_shape = jax.ShapeDtypeStruct((N_DEV * m_per, n_per), jnp.bfloat16)
    return pl.pallas_call(
        body,
        out_shape=out_shape,
        in_specs=[
            pl.BlockSpec(memory_space=pltpu.VMEM),
            pl.BlockSpec(memory_space=pltpu.VMEM),
        ],
        out_specs=pl.BlockSpec(memory_space=pltpu.VMEM),
        scratch_shapes=[
            pltpu.VMEM((m_per, n), jnp.bfloat16),
            pltpu.SemaphoreType.DMA((N_DEV,)),
            pltpu.SemaphoreType.DMA((N_DEV,)),
        ],
        compiler_params=pltpu.CompilerParams(collective_id=0),
    )(x, w_mat)


# device time: 17786 ns/iter; 1.2964x vs baseline; 1.1677x over previous
import jax
import jax.numpy as jnp
from jax import lax
from jax.experimental import pallas as pl
from jax.experimental.pallas import tpu as pltpu

N_DEV = 16
N_GRP = 4
G = N_DEV // N_GRP


def kernel(x, w_mat):
    m_per, k = x.shape
    _, n = w_mat.shape
    n_per = n // N_DEV

    def body(x_ref, w_ref, out_ref, wbuf, comm_ref, load_sems,
             send_sems, recv_sems):
        my = lax.axis_index("i")

        barrier = pltpu.get_barrier_semaphore()
        for p in range(N_DEV):
            @pl.when(my != p)
            def _():
                pl.semaphore_signal(
                    barrier, inc=1,
                    device_id=(p,), device_id_type=pl.DeviceIdType.MESH,
                )
        pl.semaphore_wait(barrier, N_DEV - 1)

        loads = []
        for jo in range(1, N_DEV + 1):
            tgt = lax.rem(my + jo, N_DEV)
            c, j = (jo - 1) // G, (jo - 1) % G
            cp = pltpu.make_async_copy(
                w_ref.at[:, pl.ds(tgt * n_per, n_per)],
                wbuf.at[c, :, pl.ds(j * n_per, n_per)],
                load_sems.at[jo - 1],
            )
            cp.start()
            loads.append(cp)

        x_mat = x_ref[:, :]
        sends = []
        for c in range(N_GRP):
            for j in range(G):
                loads[c * G + j].wait()
            grp = jnp.dot(x_mat, wbuf[c], preferred_element_type=jnp.float32)
            grp = grp * jax.nn.sigmoid(grp)
            comm_ref[c, :, :] = grp.astype(comm_ref.dtype)
            for j in range(G):
                jo = c * G + j + 1
                if jo == N_DEV:
                    out_ref[pl.ds(my * m_per, m_per), :] = (
                        comm_ref[c, :, pl.ds(j * n_per, n_per)])
                else:
                    tgt = lax.rem(my + jo, N_DEV)
                    rdma = pltpu.make_async_remote_copy(
                        src_ref=comm_ref.at[c, :, pl.ds(j * n_per, n_per)],
                        dst_ref=out_ref.at[pl.ds(my * m_per, m_per), :],
                        send_sem=send_sems.at[jo],
                        recv_sem=recv_sems.at[jo],
                        device_id=(tgt,),
                        device_id_type=pl.DeviceIdType.MESH,
                    )
                    rdma.start()
                    sends.append(rdma)

        for jo in range(1, N_DEV):
            src_dev = lax.rem(my + (N_DEV - jo), N_DEV)
            recv = pltpu.make_async_remote_copy(
                src_ref=comm_ref.at[0, :, pl.ds(0, n_per)],
                dst_ref=out_ref.at[pl.ds(src_dev * m_per, m_per), :],
                send_sem=send_sems.at[0],
                recv_sem=recv_sems.at[jo],
                device_id=(my,),
                device_id_type=pl.DeviceIdType.MESH,
            )
            recv.wait_recv()

        for rdma in sends:
            rdma.wait_send()

    out_shape = jax.ShapeDtypeStruct((N_DEV * m_per, n_per), jnp.bfloat16)
    return pl.pallas_call(
        body,
        out_shape=out_shape,
        in_specs=[
            pl.BlockSpec(memory_space=pltpu.VMEM),
            pl.BlockSpec(memory_space=pltpu.MemorySpace.HBM),
        ],
        out_specs=pl.BlockSpec(memory_space=pltpu.VMEM),
        scratch_shapes=[
            pltpu.VMEM((N_GRP, k, G * n_per), jnp.float32),
            pltpu.VMEM((N_GRP, m_per, G * n_per), jnp.bfloat16),
            pltpu.SemaphoreType.DMA((N_DEV,)),
            pltpu.SemaphoreType.DMA((N_DEV,)),
            pltpu.SemaphoreType.DMA((N_DEV,)),
        ],
        compiler_params=pltpu.CompilerParams(collective_id=0),
    )(x, pltpu.with_memory_space_constraint(w_mat, pltpu.MemorySpace.HBM))
